# Initial kernel scaffold; baseline (speedup 1.0000x reference)
import jax
import jax.numpy as jnp
from jax.experimental import pallas as pl


def kernel(boxes, scores):
    raise NotImplementedError("write your pallas kernel here")


# speedup vs baseline: 8.5125x; 8.5125x over previous
import jax
import jax.numpy as jnp
from jax.experimental import pallas as pl
from jax.experimental.pallas import tpu as pltpu

_N_TOP = 1000
_POST = 100
_IMG_H = 800.0
_IMG_W = 1333.0
_NMS_T = 0.5
_SCORE_T = 0.05


def _head_kernel(ts_ref, bc_ref, bt_ref, out_ref, m_ref):
    f32 = jnp.float32
    hi = jax.lax.Precision.HIGHEST

    bc = bc_ref[...]
    bt = bt_ref[...]

    x1c = jnp.clip(bc[:, 0:1], 0.0, _IMG_W)
    y1c = jnp.clip(bc[:, 1:2], 0.0, _IMG_H)
    x2c = jnp.clip(bc[:, 2:3], 0.0, _IMG_W)
    y2c = jnp.clip(bc[:, 3:4], 0.0, _IMG_H)
    x1r = jnp.clip(bt[0:1, :], 0.0, _IMG_W)
    y1r = jnp.clip(bt[1:2, :], 0.0, _IMG_H)
    x2r = jnp.clip(bt[2:3, :], 0.0, _IMG_W)
    y2r = jnp.clip(bt[3:4, :], 0.0, _IMG_H)

    area_c = (x2c - x1c) * (y2c - y1c)
    area_r = (x2r - x1r) * (y2r - y1r)

    wx = jnp.minimum(x2c, x2r) - jnp.maximum(x1c, x1r)
    wy = jnp.minimum(y2c, y2r) - jnp.maximum(y1c, y1r)
    inter = jnp.maximum(wx, 0.0) * jnp.maximum(wy, 0.0)
    iou = inter / (area_c + area_r - inter + 1e-9)
    i0 = jax.lax.broadcasted_iota(jnp.int32, (_N_TOP, _N_TOP), 0)
    i1 = jax.lax.broadcasted_iota(jnp.int32, (_N_TOP, _N_TOP), 1)
    m_ref[...] = ((iou > _NMS_T) & (i1 > i0)).astype(f32)

    lane = jax.lax.broadcasted_iota(jnp.int32, (1, _N_TOP), 1)

    def body(i, keep):
        row = m_ref[pl.ds(i, 1), :]
        ki = jnp.sum(keep * (lane == i).astype(f32))
        return keep * (1.0 - row * ki)

    keep = jax.lax.fori_loop(0, _N_TOP, body, jnp.ones((1, _N_TOP), f32))

    ts = ts_ref[...]
    final = ts * (ts > _SCORE_T).astype(f32) * keep

    p = (final > 0.0).astype(f32)
    z = 1.0 - p
    lt = (i0 < i1).astype(f32)
    dn = (((1,), (0,)), ((), ()))
    rank = jax.lax.dot_general(p, lt, dn, precision=hi)
    zrank = jax.lax.dot_general(z, lt, dn, precision=hi)
    num_pos = jnp.sum(p)
    orank = jnp.where(p > 0.0, rank, num_pos + zrank)
    k0 = jax.lax.broadcasted_iota(jnp.int32, (_POST, _N_TOP), 0)
    sel = (k0 == orank.astype(jnp.int32)).astype(f32)

    bcl = jnp.concatenate([x1c, y1c, x2c, y2c], axis=1)
    out_ref[:, 0:4] = jax.lax.dot_general(sel, bcl, dn, precision=hi)
    out_ref[:, 4:5] = jax.lax.dot_general(
        sel, final, (((1,), (1,)), ((), ())), precision=hi)


def kernel(boxes, scores):
    probs = jax.nn.sigmoid(scores)
    top_scores, top_idx = jax.lax.top_k(probs, _N_TOP)
    top_boxes = boxes[top_idx]
    return pl.pallas_call(
        _head_kernel,
        out_shape=jax.ShapeDtypeStruct((_POST, 5), jnp.float32),
        scratch_shapes=[pltpu.VMEM((_N_TOP, _N_TOP), jnp.float32)],
    )(top_scores.reshape(1, _N_TOP), top_boxes, top_boxes.T)


# speedup vs baseline: 9.1446x; 1.0743x over previous
import jax
import jax.numpy as jnp
from jax.experimental import pallas as pl
from jax.experimental.pallas import tpu as pltpu

_N_TOP = 1000
_N_PAD = 1024
_BLK = 128
_POST = 100
_IMG_H = 800.0
_IMG_W = 1333.0
_NMS_T = 0.5
_SCORE_T = 0.05


def _head_kernel(ts_ref, bc_ref, bt_ref, out_ref, m_ref, md_ref, keep_ref):
    f32 = jnp.float32
    hi = jax.lax.Precision.HIGHEST
    dn = (((1,), (0,)), ((), ()))

    bc = bc_ref[...]
    bt = bt_ref[...]

    x1c = jnp.clip(bc[:, 0:1], 0.0, _IMG_W)
    y1c = jnp.clip(bc[:, 1:2], 0.0, _IMG_H)
    x2c = jnp.clip(bc[:, 2:3], 0.0, _IMG_W)
    y2c = jnp.clip(bc[:, 3:4], 0.0, _IMG_H)
    x1r = jnp.clip(bt[0:1, :], 0.0, _IMG_W)
    y1r = jnp.clip(bt[1:2, :], 0.0, _IMG_H)
    x2r = jnp.clip(bt[2:3, :], 0.0, _IMG_W)
    y2r = jnp.clip(bt[3:4, :], 0.0, _IMG_H)

    area_c = (x2c - x1c) * (y2c - y1c)
    area_r = (x2r - x1r) * (y2r - y1r)

    wx = jnp.minimum(x2c, x2r) - jnp.maximum(x1c, x1r)
    wy = jnp.minimum(y2c, y2r) - jnp.maximum(y1c, y1r)
    inter = jnp.maximum(wx, 0.0) * jnp.maximum(wy, 0.0)
    iou = inter / (area_c + area_r - inter + 1e-9)
    i0 = jax.lax.broadcasted_iota(jnp.int32, (_N_PAD, _N_PAD), 0)
    i1 = jax.lax.broadcasted_iota(jnp.int32, (_N_PAD, _N_PAD), 1)
    m_ref[...] = ((iou > _NMS_T) & (i1 > i0)).astype(f32)
    keep_ref[...] = jnp.ones((1, _N_PAD), f32)
    for bs in range(0, _N_PAD, _BLK):
        md_ref[bs:bs + _BLK, :] = m_ref[bs:bs + _BLK, bs:bs + _BLK]

    lane_b = jax.lax.broadcasted_iota(jnp.int32, (1, _BLK), 1)

    for bs in range(0, _N_PAD, _BLK):

        def inner(i, kblk):
            row = md_ref[pl.ds(bs + i, 1), :]
            ki = jnp.sum(kblk * (lane_b == i).astype(f32))
            return kblk * (1.0 - row * ki)

        kblk = jax.lax.fori_loop(
            0, _BLK, inner, keep_ref[0:1, bs:bs + _BLK])
        rows = m_ref[bs:bs + _BLK, :]
        supp = jax.lax.dot_general(kblk, rows, dn)
        keep_ref[...] = keep_ref[...] * (supp <= 0.0).astype(f32)
        keep_ref[0:1, bs:bs + _BLK] = kblk

    keep = keep_ref[0:1, 0:_N_TOP]
    ts = ts_ref[...]
    final = ts * (ts > _SCORE_T).astype(f32) * keep

    p = (final > 0.0).astype(f32)
    z = 1.0 - p
    j0 = jax.lax.broadcasted_iota(jnp.int32, (_N_TOP, _N_TOP), 0)
    j1 = jax.lax.broadcasted_iota(jnp.int32, (_N_TOP, _N_TOP), 1)
    lt = (j0 < j1).astype(f32)
    rank = jax.lax.dot_general(p, lt, dn)
    zrank = jax.lax.dot_general(z, lt, dn)
    num_pos = jnp.sum(p)
    orank = jnp.where(p > 0.0, rank, num_pos + zrank)
    k0 = jax.lax.broadcasted_iota(jnp.int32, (_POST, _N_TOP), 0)
    sel = (k0 == orank.astype(jnp.int32)).astype(f32)

    bcl = jnp.concatenate(
        [x1c[:_N_TOP], y1c[:_N_TOP], x2c[:_N_TOP], y2c[:_N_TOP]], axis=1)
    out_ref[:, 0:4] = jax.lax.dot_general(sel, bcl, dn, precision=hi)
    out_ref[:, 4:5] = jax.lax.dot_general(
        sel, final, (((1,), (1,)), ((), ())), precision=hi)


def kernel(boxes, scores):
    probs = jax.nn.sigmoid(scores)
    top_scores, top_idx = jax.lax.top_k(probs, _N_TOP)
    top_boxes = boxes[top_idx]
    tb = jnp.zeros((_N_PAD, 4), jnp.float32).at[:_N_TOP].set(top_boxes)
    return pl.pallas_call(
        _head_kernel,
        out_shape=jax.ShapeDtypeStruct((_POST, 5), jnp.float32),
        scratch_shapes=[
            pltpu.VMEM((_N_PAD, _N_PAD), jnp.float32),
            pltpu.VMEM((_N_PAD, _BLK), jnp.float32),
            pltpu.VMEM((1, _N_PAD), jnp.float32),
        ],
    )(top_scores.reshape(1, _N_TOP), tb, tb.T)
